# Initial kernel scaffold; baseline (speedup 1.0000x reference)
import jax
import jax.numpy as jnp
from jax.experimental import pallas as pl


def kernel(x_0, node_idx, edge_idx, W01_0, b1_0, W10_0, b0_0, W01_1, b1_1, W10_1, b0_1, W_lin, b_lin):
    raise NotImplementedError("write your pallas kernel here")


# speedup vs baseline: 7.5523x; 7.5523x over previous
import functools

import jax
import jax.numpy as jnp
from jax import lax
from jax.experimental import pallas as pl
from jax.experimental.pallas import tpu as pltpu
from jax.experimental.pallas import tpu_sc as plsc

N_NODES = 10000
N_EDGES = 20000
NNZ = 320000
HID = 128
QW = 32

NS = 16
PER_W = NNZ // NS
CHUNK = 1000
ITERS = PER_W // CHUNK
OUT_SLICE = 2000

_mesh = plsc.VectorSubcoreMesh(core_axis_name="c", subcore_axis_name="s")


def _f32(shape):
    return jax.ShapeDtypeStruct(shape, jnp.float32)


@functools.partial(
    pl.kernel,
    out_type=(_f32((N_EDGES, 16)), _f32((N_NODES, 16))),
    mesh=_mesh,
    compiler_params=pltpu.CompilerParams(use_tc_tiling_on_sc=False),
    scratch_types=[
        pltpu.VMEM((CHUNK,), jnp.int32),
        pltpu.VMEM((CHUNK,), jnp.int32),
        pltpu.VMEM((CHUNK, 16), jnp.float32),
        pltpu.VMEM_SHARED((N_EDGES, 16), jnp.float32),
    ],
)
def _k_scalar_sums(tbl_n, tbl_e, nidx, eidx, zeros_hbm, oute, outn,
                   sidx_v, didx_v, rows_v, acc):
    c = lax.axis_index("c")
    s = lax.axis_index("s")

    @pl.when(s < N_EDGES // OUT_SLICE)
    def _():
        pltpu.sync_copy(zeros_hbm, acc.at[pl.ds(s * OUT_SLICE, OUT_SLICE)])
    plsc.subcore_barrier()

    def run(tbl, src_hbm, dst_hbm):
        @pl.loop(0, ITERS)
        def _(i):
            base = s * PER_W + i * CHUNK
            pltpu.sync_copy(src_hbm.at[pl.ds(base, CHUNK)], sidx_v)
            pltpu.sync_copy(dst_hbm.at[pl.ds(base, CHUNK)], didx_v)
            pltpu.sync_copy(tbl.at[sidx_v], rows_v)
            pltpu.sync_copy(rows_v, acc.at[didx_v], add=True)

    pl.when(c == 0)(lambda: run(tbl_n, nidx, eidx))
    pl.when(c == 1)(lambda: run(tbl_e, eidx, nidx))
    plsc.subcore_barrier()

    @pl.when(jnp.logical_and(c == 0, s < N_EDGES // OUT_SLICE))
    def _():
        sl = pl.ds(s * OUT_SLICE, OUT_SLICE)
        pltpu.sync_copy(acc.at[sl], oute.at[sl])

    @pl.when(jnp.logical_and(c == 1, s < N_NODES // OUT_SLICE))
    def _():
        sl = pl.ds(s * OUT_SLICE, OUT_SLICE)
        pltpu.sync_copy(acc.at[sl], outn.at[sl])


@functools.partial(
    pl.kernel,
    out_type=(_f32((N_EDGES, QW)), _f32((N_EDGES, QW))),
    mesh=_mesh,
    compiler_params=pltpu.CompilerParams(use_tc_tiling_on_sc=False),
    scratch_types=[
        pltpu.VMEM((CHUNK,), jnp.int32),
        pltpu.VMEM((CHUNK,), jnp.int32),
        pltpu.VMEM((CHUNK, QW), jnp.float32),
        pltpu.VMEM_SHARED((N_EDGES, QW), jnp.float32),
    ],
)
def _k_pass(tbl_a, tbl_b, sidx, didx, zeros_hbm, out_a, out_b,
            sidx_v, didx_v, rows_v, acc):
    c = lax.axis_index("c")
    s = lax.axis_index("s")

    @pl.when(s < N_EDGES // OUT_SLICE)
    def _():
        pltpu.sync_copy(zeros_hbm, acc.at[pl.ds(s * OUT_SLICE, OUT_SLICE)])
    plsc.subcore_barrier()

    def run(tbl):
        @pl.loop(0, ITERS)
        def _(i):
            base = s * PER_W + i * CHUNK
            pltpu.sync_copy(sidx.at[pl.ds(base, CHUNK)], sidx_v)
            pltpu.sync_copy(didx.at[pl.ds(base, CHUNK)], didx_v)
            pltpu.sync_copy(tbl.at[sidx_v], rows_v)
            pltpu.sync_copy(rows_v, acc.at[didx_v], add=True)

    pl.when(c == 0)(lambda: run(tbl_a))
    pl.when(c == 1)(lambda: run(tbl_b))
    plsc.subcore_barrier()

    @pl.when(jnp.logical_and(c == 0, s < N_EDGES // OUT_SLICE))
    def _():
        sl = pl.ds(s * OUT_SLICE, OUT_SLICE)
        pltpu.sync_copy(acc.at[sl], out_a.at[sl])

    @pl.when(jnp.logical_and(c == 1, s < N_EDGES // OUT_SLICE))
    def _():
        sl = pl.ds(s * OUT_SLICE, OUT_SLICE)
        pltpu.sync_copy(acc.at[sl], out_b.at[sl])


def _seg_pass(q, sidx, didx, zeros32, after=None):
    q0, q1, q2, q3 = q
    if after is not None:
        q0, q1 = lax.optimization_barrier((q0, q1, after))[:2]
    a0, a1 = _k_pass(q0, q1, sidx, didx, zeros32)
    q2, q3 = lax.optimization_barrier((q2, q3, a0))[:2]
    a2, a3 = _k_pass(q2, q3, sidx, didx, zeros32)
    return a0, a1, a2, a3


def _norm_body(ec_ref, nc_ref, te_ref, tn_ref):
    r = lax.rsqrt(jnp.maximum(ec_ref[...], 1.0))
    te_ref[...] = r * r * r
    tn_ref[...] = lax.rsqrt(jnp.maximum(nc_ref[...], 1.0))


_k_norm = pl.pallas_call(
    _norm_body,
    out_shape=(_f32((N_EDGES, 16)), _f32((N_NODES, 16))),
)


_BR = 2000


def _write_quarters(y, out_refs):
    for j, o_ref in enumerate(out_refs):
        o_ref[...] = y[:, j * QW:(j + 1) * QW]


def _in_body(x_ref, w_ref, t_ref, *out_refs):
    y = jnp.dot(x_ref[...], w_ref[...], preferred_element_type=jnp.float32,
                precision=lax.Precision.HIGHEST)
    _write_quarters(y * t_ref[:, 0:1], out_refs)


_q_specs = tuple(pl.BlockSpec((_BR, QW), lambda i: (i, 0)) for _ in range(4))
_q_shapes = tuple(_f32((N_EDGES, QW)) for _ in range(4))

_k_in = pl.pallas_call(
    _in_body,
    grid=(N_NODES // _BR,),
    in_specs=[
        pl.BlockSpec((_BR, HID), lambda i: (i, 0)),
        pl.BlockSpec((HID, HID), lambda i: (0, 0)),
        pl.BlockSpec((_BR, 16), lambda i: (i, 0)),
    ],
    out_specs=_q_specs,
    out_shape=_q_shapes,
)


def _mid_body(a0_ref, a1_ref, a2_ref, a3_ref, d_ref, b_ref, w_ref, t_ref,
              *out_refs):
    dinv = 1.0 / jnp.maximum(d_ref[:, 0:1], 1e-12)
    x = jnp.concatenate(
        [a0_ref[...], a1_ref[...], a2_ref[...], a3_ref[...]], axis=1)
    x1 = jax.nn.sigmoid(x * dinv + b_ref[...])
    y = jnp.dot(x1, w_ref[...], preferred_element_type=jnp.float32,
                precision=lax.Precision.HIGHEST)
    _write_quarters(y * t_ref[:, 0:1], out_refs)


def _make_mid(n_rows):
    return pl.pallas_call(
        _mid_body,
        grid=(n_rows // _BR,),
        in_specs=[
            *(pl.BlockSpec((_BR, QW), lambda i: (i, 0)) for _ in range(4)),
            pl.BlockSpec((_BR, 16), lambda i: (i, 0)),
            pl.BlockSpec((1, HID), lambda i: (0, 0)),
            pl.BlockSpec((HID, HID), lambda i: (0, 0)),
            pl.BlockSpec((_BR, 16), lambda i: (i, 0)),
        ],
        out_specs=_q_specs,
        out_shape=_q_shapes,
    )


_k_mid_e = _make_mid(N_EDGES)
_k_mid_n = _make_mid(N_NODES)


def _fin_body(a0_ref, a1_ref, a2_ref, a3_ref, d_ref, b_ref, wl_ref, bl_ref,
              o_ref):
    dinv = 1.0 / jnp.maximum(d_ref[:, 0:1], 1e-12)
    x = jnp.concatenate(
        [a0_ref[...], a1_ref[...], a2_ref[...], a3_ref[...]], axis=1)
    x1 = jax.nn.sigmoid(x * dinv + b_ref[...])
    m = jnp.max(x1, axis=0, keepdims=True)
    o_ref[...] = jnp.dot(m, wl_ref[...],
                         preferred_element_type=jnp.float32,
                precision=lax.Precision.HIGHEST) + bl_ref[...]


_k_fin = pl.pallas_call(
    _fin_body,
    grid=(1,),
    in_specs=[
        *(pl.BlockSpec((N_NODES, QW), lambda i: (0, 0)) for _ in range(4)),
        pl.BlockSpec((N_NODES, 16), lambda i: (0, 0)),
        pl.BlockSpec((1, HID), lambda i: (0, 0)),
        pl.BlockSpec((HID, 1), lambda i: (0, 0)),
        pl.BlockSpec((1, 1), lambda i: (0, 0)),
    ],
    out_specs=pl.BlockSpec((1, 1), lambda i: (0, 0)),
    out_shape=_f32((1, 1)),
)


def kernel(x_0, node_idx, edge_idx, W01_0, b1_0, W10_0, b0_0,
           W01_1, b1_1, W10_1, b0_1, W_lin, b_lin):
    zeros16 = jnp.zeros((OUT_SLICE, 16), jnp.float32)
    zeros32 = jnp.zeros((OUT_SLICE, QW), jnp.float32)
    ones_n = jnp.ones((N_NODES, 16), jnp.float32)
    ones_e = jnp.ones((N_EDGES, 16), jnp.float32)

    ecnt, ncnt = _k_scalar_sums(ones_n, ones_e, node_idx, edge_idx, zeros16)
    te, tn = _k_norm(ecnt, ncnt)
    d1s, d0s = _k_scalar_sums(tn, te, node_idx, edge_idx, zeros16)

    xb = _k_in(x_0, W01_0, tn)
    aa = _seg_pass(xb, node_idx, edge_idx, zeros32, after=d0s)
    xe = _k_mid_e(*aa, d1s, b1_0.reshape(1, -1), W10_0, te)
    ab = _seg_pass(xe, edge_idx, node_idx, zeros32)
    xb = _k_mid_n(*ab, d0s, b0_0.reshape(1, -1), W01_1, tn)
    aa = _seg_pass(xb, node_idx, edge_idx, zeros32)
    xe = _k_mid_e(*aa, d1s, b1_1.reshape(1, -1), W10_1, te)
    ab = _seg_pass(xe, edge_idx, node_idx, zeros32)

    out = _k_fin(*ab, d0s, b0_1.reshape(1, -1), W_lin, b_lin.reshape(1, 1))
    return out.reshape(1)
